# baseline (speedup 1.0000x reference)
import functools

import jax
import jax.numpy as jnp
from jax import lax
from jax.experimental import pallas as pl
from jax.experimental.pallas import tpu as pltpu
from jax.experimental.pallas import tpu_sc as plsc

D_MODEL = 1024
NUM_IDX = 4 * 8192

NUM_CORES = 2
NUM_SUBCORES = 16
NUM_WORKERS = NUM_CORES * NUM_SUBCORES
PER_WORKER = NUM_IDX // NUM_WORKERS
CHUNK = 8
NUM_CHUNKS = PER_WORKER // CHUNK
NBUF = 15
AHEAD = 14
NUM_GROUPS = 8

_mesh = plsc.VectorSubcoreMesh(core_axis_name="c", subcore_axis_name="s")


@functools.partial(
    pl.kernel,
    mesh=_mesh,
    out_type=jax.ShapeDtypeStruct((NUM_IDX, D_MODEL), jnp.float32),
    scratch_types=[
        pltpu.VMEM((PER_WORKER,), jnp.int32),
    ]
    + [pltpu.VMEM((CHUNK, D_MODEL), jnp.float32)] * NBUF
    + [pltpu.SemaphoreType.DMA] * (2 * NBUF),
)
def _gather_rows(idx_hbm, table_hbm, out_hbm, idx_v, *bufs_and_sems):
    bufs = bufs_and_sems[:NBUF]
    gsems = bufs_and_sems[NBUF : 2 * NBUF]
    wsems = bufs_and_sems[2 * NBUF :]

    wid = lax.axis_index("s") * NUM_CORES + lax.axis_index("c")
    base = wid * PER_WORKER
    pltpu.sync_copy(idx_hbm.at[pl.ds(base, PER_WORKER)], idx_v)

    def fire_gather(c, b):
        pltpu.async_copy(
            table_hbm.at[idx_v.at[pl.ds(c * CHUNK, CHUNK)]], bufs[b], gsems[b]
        )

    def wait_gather(c, b):
        pltpu.make_async_copy(
            table_hbm.at[idx_v.at[pl.ds(c * CHUNK, CHUNK)]], bufs[b], gsems[b]
        ).wait()

    def fire_write(c, b):
        pltpu.async_copy(
            bufs[b], out_hbm.at[pl.ds(base + c * CHUNK, CHUNK)], wsems[b]
        )

    def wait_write(c, b):
        pltpu.make_async_copy(
            bufs[b], out_hbm.at[pl.ds(base + c * CHUNK, CHUNK)], wsems[b]
        ).wait()

    for c in range(AHEAD):
        fire_gather(c, c)

    def group_body(q, carry):
        c0 = NBUF * q
        for b in range(NBUF):
            c = c0 + b
            wait_gather(c, b)
            fire_write(c, b)
            nxt = c + AHEAD
            pn = (b + AHEAD) % NBUF

            @pl.when(nxt < NUM_CHUNKS)
            def _():
                @pl.when(nxt >= NBUF)
                def _():
                    wait_write(nxt - NBUF, pn)

                fire_gather(nxt, pn)

        return carry

    lax.fori_loop(0, NUM_GROUPS, group_body, 0)

    for c in range(NUM_GROUPS * NBUF, NUM_CHUNKS):
        wait_gather(c, c % NBUF)
        fire_write(c, c % NBUF)

    for c in range(NUM_CHUNKS - NBUF, NUM_CHUNKS):
        wait_write(c, c % NBUF)


def kernel(src_seq, pos_embedding):
    flat_idx = src_seq.reshape(-1).astype(jnp.int32)
    out = _gather_rows(flat_idx, pos_embedding)
    return out.reshape(src_seq.shape + (pos_embedding.shape[1],))
